# baseline (speedup 1.0000x reference)
import jax
import jax.numpy as jnp
from jax.experimental import pallas as pl
from jax.experimental.pallas import tpu as pltpu

_HOP = 256
_SEG = 512
_KT = 512
_NT = 8
_MAIN = 4088
_TAIL = 7


def _frames_kernel(x_ref, w_ref, o_hbm, scratch, tail_buf, sems, tail_sem):
    b = pl.program_id(0)
    nb = pl.num_programs(0)
    w0 = w_ref[0, :]
    w1 = w_ref[1, :]
    tb = b % 2

    starts = [t * _KT for t in range(_NT)]
    sizes = [min(_KT, _MAIN - t * _KT) for t in range(_NT)]

    def t_copy(t, row):
        return pltpu.make_async_copy(
            scratch.at[t, pl.ds(0, sizes[t]), :],
            o_hbm.at[row, pl.ds(starts[t], sizes[t]), :],
            sems.at[t],
        )

    def tail_copy(bb, row):
        return pltpu.make_async_copy(
            tail_buf.at[bb],
            o_hbm.at[row, pl.ds(_MAIN, _TAIL), :],
            tail_sem.at[bb],
        )

    for t in range(_NT):
        k0, sz = starts[t], sizes[t]

        @pl.when(b >= 1)
        def _drain_prev(t=t):
            t_copy(t, b - 1).wait()

        scratch[t, 0:sz, 0:_HOP] = x_ref[0, k0:k0 + sz, :] * w0
        scratch[t, 0:sz, _HOP:_SEG] = x_ref[0, k0 + 1:k0 + sz + 1, :] * w1
        t_copy(t, b).start()

    @pl.when(b >= 2)
    def _drain_prev_tail():
        tail_copy(tb, b - 2).wait()

    tail_buf[tb, :, 0:_HOP] = x_ref[0, _MAIN:_MAIN + _TAIL, :] * w0
    tail_buf[tb, :, _HOP:_SEG] = x_ref[0, _MAIN + 1:_MAIN + _TAIL + 1, :] * w1
    tail_copy(tb, b).start()

    @pl.when(b == nb - 1)
    def _drain_tail():
        for t in range(_NT):
            t_copy(t, b).wait()

        @pl.when(b >= 1)
        def _():
            tail_copy(1 - tb, b - 1).wait()

        tail_copy(tb, b).wait()


def kernel(x, analysis_window):
    batch, num_samples = x.shape
    num_chunks = num_samples // _HOP
    num_frames = (num_samples - _SEG) // _HOP + 1

    x3 = x.reshape(batch, num_chunks, _HOP)
    w2 = analysis_window.reshape(2, _HOP)

    return pl.pallas_call(
        _frames_kernel,
        grid=(batch,),
        in_specs=[
            pl.BlockSpec((1, num_chunks, _HOP), lambda b: (b, 0, 0)),
            pl.BlockSpec((2, _HOP), lambda b: (0, 0)),
        ],
        out_specs=pl.BlockSpec(memory_space=pltpu.MemorySpace.HBM),
        out_shape=jax.ShapeDtypeStruct((batch, num_frames, _SEG), x.dtype),
        scratch_shapes=[
            pltpu.VMEM((_NT, _KT, _SEG), x.dtype),
            pltpu.VMEM((2, _TAIL, _SEG), x.dtype),
            pltpu.SemaphoreType.DMA((_NT,)),
            pltpu.SemaphoreType.DMA((2,)),
        ],
    )(x3, w2)


# speedup vs baseline: 1.0024x; 1.0024x over previous
import jax
import jax.numpy as jnp
from jax.experimental import pallas as pl
from jax.experimental.pallas import tpu as pltpu

_HOP = 256
_SEG = 512
_KT = 512
_NT = 8
_MAIN = 4088
_TAIL = 7


def _frames_kernel(x_ref, w_ref, o_hbm, scratch, tail_buf, sems, tail_sem):
    b = pl.program_id(0)
    nb = pl.num_programs(0)
    w0 = w_ref[0, :]
    w1 = w_ref[1, :]

    starts = [t * _KT for t in range(_NT)]
    sizes = [min(_KT, _MAIN - t * _KT) for t in range(_NT)]

    def t_copy(t, row):
        return pltpu.make_async_copy(
            scratch.at[t, pl.ds(0, sizes[t]), :],
            o_hbm.at[row, pl.ds(starts[t], sizes[t]), :],
            sems.at[t],
        )

    def tail_copy():
        return pltpu.make_async_copy(
            tail_buf,
            o_hbm.at[:, pl.ds(_MAIN, _TAIL), :],
            tail_sem,
        )

    for t in range(_NT):
        k0, sz = starts[t], sizes[t]

        @pl.when(b >= 1)
        def _drain_prev(t=t):
            t_copy(t, b - 1).wait()

        scratch[t, 0:sz, 0:_HOP] = x_ref[0, k0:k0 + sz, :] * w0
        scratch[t, 0:sz, _HOP:_SEG] = x_ref[0, k0 + 1:k0 + sz + 1, :] * w1
        t_copy(t, b).start()

    tail_buf[b, :, 0:_HOP] = x_ref[0, _MAIN:_MAIN + _TAIL, :] * w0
    tail_buf[b, :, _HOP:_SEG] = x_ref[0, _MAIN + 1:_MAIN + _TAIL + 1, :] * w1

    @pl.when(b == nb - 1)
    def _drain_tail():
        tail_copy().start()
        for t in range(_NT):
            t_copy(t, b).wait()
        tail_copy().wait()


def kernel(x, analysis_window):
    batch, num_samples = x.shape
    num_chunks = num_samples // _HOP
    num_frames = (num_samples - _SEG) // _HOP + 1

    x3 = x.reshape(batch, num_chunks, _HOP)
    w2 = analysis_window.reshape(2, _HOP)

    return pl.pallas_call(
        _frames_kernel,
        grid=(batch,),
        in_specs=[
            pl.BlockSpec((1, num_chunks, _HOP), lambda b: (b, 0, 0)),
            pl.BlockSpec((2, _HOP), lambda b: (0, 0)),
        ],
        out_specs=pl.BlockSpec(memory_space=pltpu.MemorySpace.HBM),
        out_shape=jax.ShapeDtypeStruct((batch, num_frames, _SEG), x.dtype),
        scratch_shapes=[
            pltpu.VMEM((_NT, _KT, _SEG), x.dtype),
            pltpu.VMEM((batch, _TAIL, _SEG), x.dtype),
            pltpu.SemaphoreType.DMA((_NT,)),
            pltpu.SemaphoreType.DMA,
        ],
    )(x3, w2)
